# Initial kernel scaffold; baseline (speedup 1.0000x reference)
import jax
import jax.numpy as jnp
from jax.experimental import pallas as pl


def kernel(inputs, W0, W1, W2, W3, W4, W5):
    raise NotImplementedError("write your pallas kernel here")


# speedup vs baseline: 1.5478x; 1.5478x over previous
import functools

import jax
import jax.numpy as jnp
from jax import lax
from jax.experimental import pallas as pl
from jax.experimental.pallas import tpu as pltpu
from jax.experimental.pallas import tpu_sc as plsc

DIM = 768
NUM_TABLES = 6


@functools.lru_cache(maxsize=None)
def _make_gather(B: int, D: int):
    info = plsc.get_sparse_core_info()
    NC, NS = info.num_cores, info.num_subcores
    NW = NC * NS
    assert B % (8 * NW) == 0
    b_per_w = B // NW

    mesh = plsc.VectorSubcoreMesh(core_axis_name="c", subcore_axis_name="s")

    @functools.partial(
        pl.kernel,
        mesh=mesh,
        out_type=[jax.ShapeDtypeStruct((B, D), jnp.float32)] * NUM_TABLES,
        scratch_types=[
            pltpu.VMEM((b_per_w,), jnp.int32),
            pltpu.VMEM((b_per_w, D), jnp.float32),
            pltpu.SemaphoreType.DMA,
        ],
    )
    def gather6(idx_hbm, t0, t1, t2, t3, t4, t5,
                o0, o1, o2, o3, o4, o5, idx_v, rows_v, sem):
        wid = lax.axis_index("s") * NC + lax.axis_index("c")
        base = wid * b_per_w
        pltpu.sync_copy(idx_hbm.at[pl.ds(base, b_per_w)], idx_v)
        for table, out in ((t0, o0), (t1, o1), (t2, o2),
                           (t3, o3), (t4, o4), (t5, o5)):
            pltpu.async_copy(table.at[idx_v], rows_v, sem).wait()
            pltpu.sync_copy(rows_v, out.at[pl.ds(base, b_per_w)])

    return gather6


def kernel(inputs, W0, W1, W2, W3, W4, W5):
    batch, seq = inputs.shape
    flat_idx = inputs.reshape(-1).astype(jnp.int32)
    outs = _make_gather(batch * seq, DIM)(flat_idx, W0, W1, W2, W3, W4, W5)
    ve = [o.reshape(batch, seq, DIM) for o in outs]
    return tuple(ve + ve[::-1])


# speedup vs baseline: 1.5717x; 1.0154x over previous
import functools

import jax
import jax.numpy as jnp
from jax import lax
from jax.experimental import pallas as pl
from jax.experimental.pallas import tpu as pltpu
from jax.experimental.pallas import tpu_sc as plsc

DIM = 768
NUM_TABLES = 6
CHUNK = 64
NBUF = 2


@functools.lru_cache(maxsize=None)
def _make_gather(B: int, D: int):
    info = plsc.get_sparse_core_info()
    NC, NS = info.num_cores, info.num_subcores
    NW = NC * NS
    assert B % (8 * NW) == 0
    b_per_w = B // NW
    assert b_per_w % CHUNK == 0
    n_chunks = b_per_w // CHUNK
    n_steps = NUM_TABLES * n_chunks

    mesh = plsc.VectorSubcoreMesh(core_axis_name="c", subcore_axis_name="s")

    @functools.partial(
        pl.kernel,
        mesh=mesh,
        out_type=[jax.ShapeDtypeStruct((B, D), jnp.float32)] * NUM_TABLES,
        scratch_types=(
            [pltpu.VMEM((n_chunks, CHUNK), jnp.int32)]
            + [pltpu.VMEM((CHUNK, D), jnp.float32)] * NBUF
            + [pltpu.SemaphoreType.DMA] * (2 * NBUF)
        ),
    )
    def gather6(idx_hbm, t0, t1, t2, t3, t4, t5,
                o0, o1, o2, o3, o4, o5, idx_v, *bufs_and_sems):
        rows = bufs_and_sems[:NBUF]
        gsem = bufs_and_sems[NBUF:2 * NBUF]
        ssem = bufs_and_sems[2 * NBUF:]
        tables = (t0, t1, t2, t3, t4, t5)
        outs = (o0, o1, o2, o3, o4, o5)
        wid = lax.axis_index("s") * NC + lax.axis_index("c")
        base = wid * b_per_w
        pltpu.sync_copy(
            idx_hbm.at[pl.ds(wid * n_chunks, n_chunks)], idx_v)

        def start_gather(s):
            t, c = divmod(s, n_chunks)
            b = s % NBUF
            return pltpu.async_copy(
                tables[t].at[idx_v.at[c]], rows[b], gsem[b])

        def start_scatter(s):
            t, c = divmod(s, n_chunks)
            b = s % NBUF
            return pltpu.async_copy(
                rows[b], outs[t].at[pl.ds(base + c * CHUNK, CHUNK)], ssem[b])

        g_h = [None] * n_steps
        s_h = [None] * n_steps
        for s in range(NBUF):
            g_h[s] = start_gather(s)
        for s in range(n_steps):
            g_h[s].wait()
            s_h[s] = start_scatter(s)
            nxt = s + NBUF
            if nxt < n_steps:
                s_h[s].wait()
                g_h[nxt] = start_gather(nxt)
        for s in range(n_steps - NBUF, n_steps):
            s_h[s].wait()

    return gather6


def kernel(inputs, W0, W1, W2, W3, W4, W5):
    batch, seq = inputs.shape
    flat_idx = inputs.reshape(-1, CHUNK).astype(jnp.int32)
    outs = _make_gather(batch * seq, DIM)(flat_idx, W0, W1, W2, W3, W4, W5)
    ve = [o.reshape(batch, seq, DIM) for o in outs]
    return tuple(ve + ve[::-1])


# speedup vs baseline: 2.0848x; 1.3265x over previous
import functools

import jax
import jax.numpy as jnp
from jax import lax
from jax.experimental import pallas as pl
from jax.experimental.pallas import tpu as pltpu
from jax.experimental.pallas import tpu_sc as plsc

DIM = 768
NUM_TABLES = 6
CHUNK = 64
NBUF = 2


@functools.lru_cache(maxsize=None)
def _make_gather(B: int, D: int):
    info = plsc.get_sparse_core_info()
    NC, NS = info.num_cores, info.num_subcores
    NW = NC * NS
    assert B % (8 * NW) == 0
    b_per_w = B // NW
    assert b_per_w % CHUNK == 0
    n_chunks = b_per_w // CHUNK
    n_steps = NUM_TABLES * n_chunks

    mesh = plsc.VectorSubcoreMesh(core_axis_name="c", subcore_axis_name="s")

    @functools.partial(
        pl.kernel,
        mesh=mesh,
        out_type=[jax.ShapeDtypeStruct((B, D), jnp.float32)] * (2 * NUM_TABLES),
        scratch_types=(
            [pltpu.VMEM((n_chunks, CHUNK), jnp.int32)]
            + [pltpu.VMEM((CHUNK, D), jnp.float32)] * NBUF
            + [pltpu.SemaphoreType.DMA] * (2 * NBUF)
        ),
    )
    def gather6(idx_hbm, t0, t1, t2, t3, t4, t5, *rest):
        outs = rest[:2 * NUM_TABLES]
        idx_v = rest[2 * NUM_TABLES]
        rows = rest[2 * NUM_TABLES + 1:2 * NUM_TABLES + 1 + NBUF]
        sems = rest[2 * NUM_TABLES + 1 + NBUF:]
        gsem = sems[:NBUF]
        ssem = sems[NBUF:]
        tables = (t0, t1, t2, t3, t4, t5)
        wid = lax.axis_index("s") * NC + lax.axis_index("c")
        base = wid * b_per_w
        pltpu.sync_copy(
            idx_hbm.at[pl.ds(wid * n_chunks, n_chunks)], idx_v)

        def start_gather(s):
            t, c = divmod(s, n_chunks)
            b = s % NBUF
            return pltpu.async_copy(
                tables[t].at[idx_v.at[c]], rows[b], gsem[b])

        def start_scatters(s):
            t, c = divmod(s, n_chunks)
            b = s % NBUF
            dst = pl.ds(base + c * CHUNK, CHUNK)
            return (
                pltpu.async_copy(rows[b], outs[t].at[dst], ssem[b]),
                pltpu.async_copy(rows[b], outs[11 - t].at[dst], ssem[b]),
            )

        g_h = [None] * n_steps
        s_h = [None] * n_steps
        for s in range(NBUF):
            g_h[s] = start_gather(s)
        for s in range(n_steps):
            g_h[s].wait()
            s_h[s] = start_scatters(s)
            nxt = s + NBUF
            if nxt < n_steps:
                for h in s_h[s]:
                    h.wait()
                g_h[nxt] = start_gather(nxt)
        for s in range(n_steps - NBUF, n_steps):
            for h in s_h[s]:
                h.wait()

    return gather6


def kernel(inputs, W0, W1, W2, W3, W4, W5):
    batch, seq = inputs.shape
    flat_idx = inputs.reshape(-1, CHUNK).astype(jnp.int32)
    outs = _make_gather(batch * seq, DIM)(flat_idx, W0, W1, W2, W3, W4, W5)
    return tuple(o.reshape(batch, seq, DIM) for o in outs)
